# baseline (speedup 1.0000x reference)
import math
import functools

import jax
import jax.numpy as jnp
from jax import lax
from jax.experimental import pallas as pl
from jax.experimental.pallas import tpu as pltpu

N = 10000
E = 320000
G = 128
XD = 78
XV = 3904
XF = 881
XS = 61


def _linear(x, W, b):
    return x @ W.T + b


def _ln(x, g, b, eps=1e-5):
    m = jnp.mean(x, axis=-1, keepdims=True)
    v = jnp.var(x, axis=-1, keepdims=True)
    return (x - m) / jnp.sqrt(v + eps) * g + b


def _bn(x, g, b, eps=1e-5):
    m = jnp.mean(x, axis=0)
    v = jnp.var(x, axis=0)
    return (x - m) / jnp.sqrt(v + eps) * g + b


def _mha(x, p, nh):
    B, L, D = x.shape
    hd = D // nh

    def split(t):
        return jnp.transpose(t.reshape(B, L, nh, hd), (0, 2, 1, 3))

    q = split(_linear(x, p['Wq'], p['bq']))
    k = split(_linear(x, p['Wk'], p['bk']))
    v = split(_linear(x, p['Wv'], p['bv']))
    s = jnp.einsum('bhqd,bhkd->bhqk', q, k) / math.sqrt(hd)
    a = jax.nn.softmax(s, axis=-1)
    o = jnp.einsum('bhqk,bhkd->bhqd', a, v)
    o = jnp.transpose(o, (0, 2, 1, 3)).reshape(B, L, D)
    return _linear(o, p['Wo'], p['bo'])


def _transformer(x, p, nh):
    tok = _linear(x, p['emb_W'], p['emb_b'])
    pe = jnp.sin(jnp.arange(x.shape[1], dtype=jnp.float32))[None, :, None]
    h = _ln(tok + (x + pe), p['embln_g'], p['embln_b'], eps=1e-12)
    a = _ln(h, p['ln1_g'], p['ln1_b'])
    h = h + _mha(a, p, nh)
    f = _ln(h, p['ln2_g'], p['ln2_b'])
    f = _linear(jax.nn.gelu(_linear(f, p['ff1_W'], p['ff1_b']), approximate=False),
                p['ff2_W'], p['ff2_b'])
    return h + f


def _hconv(x, ei, W, b, n):
    node, he = ei[0], ei[1]
    xl = x @ W.T
    nnz = ei.shape[1]
    D = jax.ops.segment_sum(jnp.ones((nnz,), jnp.float32), node, n)
    Dinv = jnp.where(D > 0, 1.0 / D, 0.0)
    Bdeg = jax.ops.segment_sum(jnp.ones((nnz,), jnp.float32), he, n)
    Binv = jnp.where(Bdeg > 0, 1.0 / Bdeg, 0.0)
    m = jax.ops.segment_sum(xl[node] * Binv[he][:, None], he, n)
    out = jax.ops.segment_sum(m[he] * Dinv[node][:, None], node, n)
    return out + b


def _gcn(x, ei, W, b, n):
    xl = x @ W.T
    sl = jnp.arange(n, dtype=ei.dtype)
    row = jnp.concatenate([ei[0], sl])
    col = jnp.concatenate([ei[1], sl])
    deg = jax.ops.segment_sum(jnp.ones((row.shape[0],), jnp.float32), col, n)
    dis = jnp.where(deg > 0, deg ** -0.5, 0.0)
    norm = dis[row] * dis[col]
    out = jax.ops.segment_sum(xl[row] * norm[:, None], col, n)
    return out + b


def _head_body(xc_ref, w3_ref, b3_ref, g3_ref, bb3_ref, w4_ref, b4_ref,
               g4_ref, bb4_ref, w5_ref, b5_ref, wo_ref, bo_ref, out_ref):
    eps = 1e-5
    xc = xc_ref[...]
    h = jnp.maximum(xc @ w3_ref[...].T + b3_ref[...], 0.0)
    m = jnp.mean(h, axis=0, keepdims=True)
    v = jnp.mean((h - m) ** 2, axis=0, keepdims=True)
    h = (h - m) / jnp.sqrt(v + eps) * g3_ref[...] + bb3_ref[...]
    h = jnp.maximum(h @ w4_ref[...].T + b4_ref[...], 0.0)
    m = jnp.mean(h, axis=0, keepdims=True)
    v = jnp.mean((h - m) ** 2, axis=0, keepdims=True)
    h = (h - m) / jnp.sqrt(v + eps) * g4_ref[...] + bb4_ref[...]
    h = jnp.maximum(h @ w5_ref[...].T + b5_ref[...], 0.0)
    out_ref[...] = h @ wo_ref[...].T + bo_ref[...]


def _head(xc, p):
    return pl.pallas_call(
        _head_body,
        out_shape=jax.ShapeDtypeStruct((xc.shape[0], 2), jnp.float32),
    )(xc, p['fc_3_W'], p['fc_3_b'], p['n256_g'], p['n256_b'],
      p['fc_4_W'], p['fc_4_b'], p['n128_g'], p['n128_b'],
      p['fc_5_W'], p['fc_5_b'], p['out_W'], p['out_b'])


def kernel(x, edge_index, batch, fingerprint, seqbefore, seqafter, variant, params):
    p = params
    h = jax.nn.relu(_hconv(x, edge_index, p['hc1_W'], p['hc1_b'], N))
    h = jax.nn.relu(_hconv(h, edge_index, p['hc2_W'], p['hc2_b'], N))
    h = jax.nn.relu(_gcn(h, edge_index, p['gcn_W'], p['gcn_b'], N))
    gmax = jax.ops.segment_max(h, batch, G)
    cnt = jnp.clip(jax.ops.segment_sum(jnp.ones((N,), jnp.float32), batch, G), 1.0, None)
    gmean = jax.ops.segment_sum(h, batch, G) / cnt[:, None]
    xg = jnp.concatenate([gmax, gmean], axis=1)
    xg = jax.nn.relu(_linear(xg, p['fc_g1_W'], p['fc_g1_b']))
    xg = _linear(xg, p['fc_g2_W'], p['fc_g2_b'])
    tf = _transformer(fingerprint[..., None], p['tf'], 8)
    flat_fc = jax.nn.relu(_linear(tf.reshape(tf.shape[0], -1), p['flat_fc_W'], p['flat_fc_b']))
    flat_fc = _bn(flat_fc, p['nxf_g'], p['nxf_b'])
    tsb = _transformer(seqbefore, p['tb'], 4)
    tsa = _transformer(seqafter, p['ta'], 4)
    fxsb = jax.nn.relu(_linear(tsb.reshape(tsb.shape[0], -1), p['flat_xs_W'], p['flat_xs_b']))
    fxsa = jax.nn.relu(_linear(tsa.reshape(tsa.shape[0], -1), p['flat_xs_W'], p['flat_xs_b']))
    fv = jax.nn.relu(_linear(variant, p['fc_xv_W'], p['fc_xv_b']))
    fv = _bn(fv, p['nxv_g'], p['nxv_b'])
    c1 = jnp.concatenate([flat_fc, fv], axis=1)
    c1 = _bn(jax.nn.relu(_linear(c1, p['fc2_1_W'], p['fc2_1_b'])), p['n1024_g'], p['n1024_b'])
    c2 = jnp.concatenate([fxsb, fxsa], axis=1)
    c2 = _bn(jax.nn.relu(_linear(c2, p['fc2_2_W'], p['fc2_2_b'])), p['n128_g'], p['n128_b'])
    xc = jnp.concatenate([xg, c1, c2], axis=1)
    return _head(xc, p)


# speedup vs baseline: 2.9329x; 2.9329x over previous
import math
import functools

import jax
import jax.numpy as jnp
from jax import lax
from jax.experimental import pallas as pl
from jax.experimental.pallas import tpu as pltpu
from jax.experimental.pallas import tpu_sc as plsc

N = 10000
E = 320000
G = 128
XD = 78
XV = 3904
XF = 881
XS = 61

NC, NS, L = 2, 16, 16
NP = 10240
HALF = NP // NC
AROWS = 5248
TRASH = HALF
EPT = E // NS
KCH = 80
NCHE = EPT // KCH
ARPT = AROWS // NS
ORPT = HALF // NS
OCH = 64

_mesh = plsc.VectorSubcoreMesh(core_axis_name="c", subcore_axis_name="s")


def _zero_vmem(ref, nrows, ncols):
    def row(i, _):
        def col(j, _):
            ref[i, pl.ds(j * L, L)] = jnp.zeros((L,), jnp.float32)
            return 0
        return lax.fori_loop(0, ncols // L, col, 0)
    lax.fori_loop(0, nrows, row, 0)


def _fill_vmem(ref, nrows, ncols, val):
    def row(i, _):
        def col(j, _):
            ref[i, pl.ds(j * L, L)] = jnp.full((L,), val, jnp.float32)
            return 0
        return lax.fori_loop(0, ncols // L, col, 0)
    lax.fori_loop(0, nrows, row, 0)


def _localize(idx_ref, lo):
    def row(c, _):
        def col(j, _):
            d = idx_ref[c, pl.ds(j * L, L)]
            m = (d >= lo) & (d < lo + HALF)
            idx_ref[c, pl.ds(j * L, L)] = jnp.where(m, d - lo, TRASH)
            return 0
        return lax.fori_loop(0, KCH // L, col, 0)
    lax.fori_loop(0, NCHE, row, 0)


def _zero_acc(acc, zb_v, sid):
    base = sid * ARPT
    for z in range(ARPT // OCH):
        pltpu.sync_copy(zb_v, acc.at[pl.ds(base + z * OCH, OCH)])
    pltpu.sync_copy(zb_v.at[pl.ds(0, ARPT % OCH)],
                    acc.at[pl.ds(base + (ARPT // OCH) * OCH, ARPT % OCH)])


def _copy_out(acc, bb_v, out_slice_fn, cid, sid):
    for z in range(ORPT // OCH):
        r = sid * ORPT + z * OCH
        pltpu.sync_copy(acc.at[pl.ds(r, OCH)], bb_v)
        pltpu.sync_copy(bb_v, out_slice_fn(cid * HALF + r))


@functools.partial(
    pl.kernel, mesh=_mesh,
    out_type=jax.ShapeDtypeStruct((2, NP, 128), jnp.float32),
    scratch_types=[
        pltpu.VMEM((NCHE, KCH), jnp.int32),
        pltpu.VMEM((KCH, 128), jnp.float32),
        pltpu.VMEM((OCH, 128), jnp.float32),
        pltpu.VMEM_SHARED((AROWS, 128), jnp.float32),
    ],
)
def _sc_counts(ei0_hbm, ei1_hbm, out_hbm, idx_v, ones_v, zb_v, acc):
    cid = lax.axis_index("c")
    sid = lax.axis_index("s")
    lo = cid * HALF
    _fill_vmem(ones_v, KCH, 128, 1.0)
    _zero_vmem(zb_v, OCH, 128)
    for which, ei_hbm in ((0, ei0_hbm), (1, ei1_hbm)):
        _zero_acc(acc, zb_v, sid)
        plsc.subcore_barrier()
        pltpu.sync_copy(ei_hbm.at[sid], idx_v)
        _localize(idx_v, lo)

        def body(c, _):
            pltpu.sync_copy(ones_v, acc.at[idx_v.at[c]], add=True)
            return 0
        lax.fori_loop(0, NCHE, body, 0)
        plsc.subcore_barrier()
        _copy_out(acc, zb_v,
                  lambda r, w=which: out_hbm.at[w, pl.ds(r, OCH)], cid, sid)
        plsc.subcore_barrier()


@functools.lru_cache(maxsize=None)
def _sc_propagate(dp):
    @functools.partial(
        pl.kernel, mesh=_mesh,
        out_type=jax.ShapeDtypeStruct((NP, dp), jnp.float32),
        scratch_types=[
            pltpu.VMEM((NCHE, KCH), jnp.int32),
            pltpu.VMEM((NCHE, KCH), jnp.int32),
            pltpu.VMEM((KCH, dp), jnp.float32),
            pltpu.VMEM((OCH, dp), jnp.float32),
            pltpu.VMEM_SHARED((AROWS, dp), jnp.float32),
            pltpu.SemaphoreType.DMA,
        ],
    )
    def k(xsrc_hbm, src_hbm, dst_hbm, out_hbm, sidx_v, didx_v, rows_v, zb_v,
          acc, sem):
        cid = lax.axis_index("c")
        sid = lax.axis_index("s")
        lo = cid * HALF
        _zero_vmem(zb_v, OCH, dp)
        _zero_acc(acc, zb_v, sid)
        plsc.subcore_barrier()

        pltpu.sync_copy(src_hbm.at[sid], sidx_v)
        pltpu.sync_copy(dst_hbm.at[sid], didx_v)
        _localize(didx_v, lo)

        def body(c, _):
            pltpu.async_copy(xsrc_hbm.at[sidx_v.at[c]], rows_v, sem).wait()
            pltpu.sync_copy(rows_v, acc.at[didx_v.at[c]], add=True)
            return 0
        lax.fori_loop(0, NCHE, body, 0)

        plsc.subcore_barrier()
        _copy_out(acc, zb_v, lambda r: out_hbm.at[pl.ds(r, OCH)], cid, sid)
    return k


_MBLK = 1024


def _tc_matmul(x, w):
    n, k = x.shape
    o = w.shape[0]

    def body(x_ref, w_ref, o_ref):
        o_ref[...] = lax.dot_general(
            x_ref[...], w_ref[...], (((1,), (1,)), ((), ())),
            preferred_element_type=jnp.float32)

    return pl.pallas_call(
        body,
        grid=(n // _MBLK,),
        in_specs=[pl.BlockSpec((_MBLK, k), lambda i: (i, 0)),
                  pl.BlockSpec((o, k), lambda i: (0, 0))],
        out_specs=pl.BlockSpec((_MBLK, o), lambda i: (i, 0)),
        out_shape=jax.ShapeDtypeStruct((n, o), jnp.float32),
    )(x, w)


def _tc_combine(t, cnt, mode, bias=None, xl=None):
    dp = xl.shape[1] if t is None else t.shape[1]
    nin = []
    specs = []
    if t is not None:
        nin.append(t)
        specs.append(pl.BlockSpec((_MBLK, dp), lambda i: (i, 0)))
    nin.append(cnt)
    specs.append(pl.BlockSpec((_MBLK, L), lambda i: (i, 0)))
    if xl is not None:
        nin.append(xl)
        specs.append(pl.BlockSpec((_MBLK, dp), lambda i: (i, 0)))
    if bias is not None:
        nin.append(bias.reshape(1, dp))
        specs.append(pl.BlockSpec((1, dp), lambda i: (0, 0)))

    def body(*refs):
        refs = list(refs)
        out_ref = refs.pop()
        tv = refs.pop(0)[...] if t is not None else None
        cv = refs.pop(0)[...][:, 0:1]
        xv = refs.pop(0)[...] if xl is not None else None
        bv = refs.pop(0)[...] if bias is not None else None
        if mode == 'binv':
            out_ref[...] = jnp.where(cv > 0, tv / jnp.where(cv > 0, cv, 1.0), 0.0)
        elif mode == 'dinv':
            s = jnp.where(cv > 0, tv / jnp.where(cv > 0, cv, 1.0), 0.0)
            out_ref[...] = jnp.maximum(s + bv, 0.0)
        elif mode == 'dis':
            out_ref[...] = lax.rsqrt(cv + 1.0) * xv
        elif mode == 'gcn':
            d = lax.rsqrt(cv + 1.0)
            out_ref[...] = jnp.maximum(d * tv + d * d * xv + bv, 0.0)

    return pl.pallas_call(
        body,
        grid=(NP // _MBLK,),
        in_specs=specs,
        out_specs=pl.BlockSpec((_MBLK, dp), lambda i: (i, 0)),
        out_shape=jax.ShapeDtypeStruct((NP, dp), jnp.float32),
    )(*nin)


def _pad_w(w, o_t, k_t):
    return jnp.pad(w, ((0, o_t - w.shape[0]), (0, k_t - w.shape[1])))


def _head_body(xc_ref, w3_ref, b3_ref, g3_ref, bb3_ref, w4_ref, b4_ref,
               g4_ref, bb4_ref, w5_ref, b5_ref, wo_ref, bo_ref, out_ref):
    eps = 1e-5
    xc = xc_ref[...]
    h = jnp.maximum(xc @ w3_ref[...].T + b3_ref[...], 0.0)
    m = jnp.mean(h, axis=0, keepdims=True)
    v = jnp.mean((h - m) ** 2, axis=0, keepdims=True)
    h = (h - m) / jnp.sqrt(v + eps) * g3_ref[...] + bb3_ref[...]
    h = jnp.maximum(h @ w4_ref[...].T + b4_ref[...], 0.0)
    m = jnp.mean(h, axis=0, keepdims=True)
    v = jnp.mean((h - m) ** 2, axis=0, keepdims=True)
    h = (h - m) / jnp.sqrt(v + eps) * g4_ref[...] + bb4_ref[...]
    h = jnp.maximum(h @ w5_ref[...].T + b5_ref[...], 0.0)
    out_ref[...] = h @ wo_ref[...].T + bo_ref[...]


def _head(xc, p):
    return pl.pallas_call(
        _head_body,
        out_shape=jax.ShapeDtypeStruct((xc.shape[0], 2), jnp.float32),
    )(xc, p['fc_3_W'], p['fc_3_b'], p['n256_g'], p['n256_b'],
      p['fc_4_W'], p['fc_4_b'], p['n128_g'], p['n128_b'],
      p['fc_5_W'], p['fc_5_b'], p['out_W'], p['out_b'])


def _linear(x, W, b):
    return x @ W.T + b


def _ln(x, g, b, eps=1e-5):
    m = jnp.mean(x, axis=-1, keepdims=True)
    v = jnp.var(x, axis=-1, keepdims=True)
    return (x - m) / jnp.sqrt(v + eps) * g + b


def _bn(x, g, b, eps=1e-5):
    m = jnp.mean(x, axis=0)
    v = jnp.var(x, axis=0)
    return (x - m) / jnp.sqrt(v + eps) * g + b


def _mha(x, p, nh):
    B, Lq, D = x.shape
    hd = D // nh

    def split(t):
        return jnp.transpose(t.reshape(B, Lq, nh, hd), (0, 2, 1, 3))

    q = split(_linear(x, p['Wq'], p['bq']))
    k = split(_linear(x, p['Wk'], p['bk']))
    v = split(_linear(x, p['Wv'], p['bv']))
    s = jnp.einsum('bhqd,bhkd->bhqk', q, k) / math.sqrt(hd)
    a = jax.nn.softmax(s, axis=-1)
    o = jnp.einsum('bhqk,bhkd->bhqd', a, v)
    o = jnp.transpose(o, (0, 2, 1, 3)).reshape(B, Lq, D)
    return _linear(o, p['Wo'], p['bo'])


def _transformer(x, p, nh):
    tok = _linear(x, p['emb_W'], p['emb_b'])
    pe = jnp.sin(jnp.arange(x.shape[1], dtype=jnp.float32))[None, :, None]
    h = _ln(tok + (x + pe), p['embln_g'], p['embln_b'], eps=1e-12)
    a = _ln(h, p['ln1_g'], p['ln1_b'])
    h = h + _mha(a, p, nh)
    f = _ln(h, p['ln2_g'], p['ln2_b'])
    f = _linear(jax.nn.gelu(_linear(f, p['ff1_W'], p['ff1_b']), approximate=False),
                p['ff2_W'], p['ff2_b'])
    return h + f


def kernel(x, edge_index, batch, fingerprint, seqbefore, seqafter, variant, params):
    p = params
    src = edge_index[0].reshape(NS, NCHE, KCH)
    dst = edge_index[1].reshape(NS, NCHE, KCH)

    cparts = _sc_counts(src, dst)
    cnt_node = cparts[0, :, :L]
    cnt_he = cparts[1, :, :L]

    xpad = jnp.pad(x, ((0, NP - N), (0, 0)))

    w1 = _pad_w(p['hc1_W'], 128, XD)
    xl1 = _tc_matmul(xpad, w1)
    s1 = _sc_propagate(128)(xl1, src, dst)
    m1 = _tc_combine(s1, cnt_he, 'binv')
    t1 = _sc_propagate(128)(m1, dst, src)
    h1 = _tc_combine(t1, cnt_node, 'dinv',
                     bias=jnp.pad(p['hc1_b'], (0, 50)))

    w2 = _pad_w(p['hc2_W'], 384, 128)
    xl2 = _tc_matmul(h1, w2)
    b2 = jnp.pad(p['hc2_b'], (0, 72))
    h2panels = []
    for c0 in range(0, 384, 128):
        s2 = _sc_propagate(128)(xl2[:, c0:c0 + 128], src, dst)
        m2 = _tc_combine(s2, cnt_he, 'binv')
        t2 = _sc_propagate(128)(m2, dst, src)
        h2panels.append(_tc_combine(t2, cnt_node, 'dinv', bias=b2[c0:c0 + 128]))
    h2 = jnp.concatenate(h2panels, axis=1)

    w3 = _pad_w(p['gcn_W'], 896, 384)
    xl3 = _tc_matmul(h2, w3)
    b3 = jnp.pad(p['gcn_b'], (0, 116))
    h3panels = []
    for c0 in range(0, 896, 128):
        xl3p = xl3[:, c0:c0 + 128]
        zp = _tc_combine(None, cnt_he, 'dis', xl=xl3p)
        t3 = _sc_propagate(128)(zp, src, dst)
        h3panels.append(_tc_combine(t3, cnt_he, 'gcn', bias=b3[c0:c0 + 128],
                                    xl=xl3p))
    h = jnp.concatenate(h3panels, axis=1)[:N, :780]

    gmax = jax.ops.segment_max(h, batch, G)
    cnt = jnp.clip(jax.ops.segment_sum(jnp.ones((N,), jnp.float32), batch, G), 1.0, None)
    gmean = jax.ops.segment_sum(h, batch, G) / cnt[:, None]
    xg = jnp.concatenate([gmax, gmean], axis=1)
    xg = jax.nn.relu(_linear(xg, p['fc_g1_W'], p['fc_g1_b']))
    xg = _linear(xg, p['fc_g2_W'], p['fc_g2_b'])
    tf = _transformer(fingerprint[..., None], p['tf'], 8)
    flat_fc = jax.nn.relu(_linear(tf.reshape(tf.shape[0], -1), p['flat_fc_W'], p['flat_fc_b']))
    flat_fc = _bn(flat_fc, p['nxf_g'], p['nxf_b'])
    tsb = _transformer(seqbefore, p['tb'], 4)
    tsa = _transformer(seqafter, p['ta'], 4)
    fxsb = jax.nn.relu(_linear(tsb.reshape(tsb.shape[0], -1), p['flat_xs_W'], p['flat_xs_b']))
    fxsa = jax.nn.relu(_linear(tsa.reshape(tsa.shape[0], -1), p['flat_xs_W'], p['flat_xs_b']))
    fv = jax.nn.relu(_linear(variant, p['fc_xv_W'], p['fc_xv_b']))
    fv = _bn(fv, p['nxv_g'], p['nxv_b'])
    c1 = jnp.concatenate([flat_fc, fv], axis=1)
    c1 = _bn(jax.nn.relu(_linear(c1, p['fc2_1_W'], p['fc2_1_b'])), p['n1024_g'], p['n1024_b'])
    c2 = jnp.concatenate([fxsb, fxsa], axis=1)
    c2 = _bn(jax.nn.relu(_linear(c2, p['fc2_2_W'], p['fc2_2_b'])), p['n128_g'], p['n128_b'])
    xc = jnp.concatenate([xg, c1, c2], axis=1)
    return _head(xc, p)


# speedup vs baseline: 3.2620x; 1.1122x over previous
import math
import functools

import jax
import jax.numpy as jnp
from jax import lax
from jax.experimental import pallas as pl
from jax.experimental.pallas import tpu as pltpu
from jax.experimental.pallas import tpu_sc as plsc

N = 10000
E = 320000
G = 128
XD = 78
XV = 3904
XF = 881
XS = 61

NC, NS, L = 2, 16, 16
NP = 10240
HALF = NP // NC
AROWS = 5248
TRASH = HALF
EPT = E // NS
KCH = 80
NST = 2
SCH = 125
NCHE = NST * SCH
ARPT = AROWS // NS
ORPT = HALF // NS
OCH = 64

_mesh = plsc.VectorSubcoreMesh(core_axis_name="c", subcore_axis_name="s")


def _zero_vmem(ref, nrows, ncols):
    def row(i, _):
        def col(j, _):
            ref[i, pl.ds(j * L, L)] = jnp.zeros((L,), jnp.float32)
            return 0
        return lax.fori_loop(0, ncols // L, col, 0)
    lax.fori_loop(0, nrows, row, 0)


def _fill_vmem(ref, nrows, ncols, val):
    def row(i, _):
        def col(j, _):
            ref[i, pl.ds(j * L, L)] = jnp.full((L,), val, jnp.float32)
            return 0
        return lax.fori_loop(0, ncols // L, col, 0)
    lax.fori_loop(0, nrows, row, 0)


def _localize(idx_ref, lo):
    def row(c, _):
        def col(j, _):
            d = idx_ref[c, pl.ds(j * L, L)]
            m = (d >= lo) & (d < lo + HALF)
            idx_ref[c, pl.ds(j * L, L)] = jnp.where(m, d - lo, TRASH)
            return 0
        return lax.fori_loop(0, KCH // L, col, 0)
    lax.fori_loop(0, SCH, row, 0)


def _zero_acc(acc, zb_v, sid):
    base = sid * ARPT
    for z in range(ARPT // OCH):
        pltpu.sync_copy(zb_v, acc.at[pl.ds(base + z * OCH, OCH)])
    pltpu.sync_copy(zb_v.at[pl.ds(0, ARPT % OCH)],
                    acc.at[pl.ds(base + (ARPT // OCH) * OCH, ARPT % OCH)])


def _copy_out(acc, bb_v, out_slice_fn, cid, sid):
    for z in range(ORPT // OCH):
        r = sid * ORPT + z * OCH
        pltpu.sync_copy(acc.at[pl.ds(r, OCH)], bb_v)
        pltpu.sync_copy(bb_v, out_slice_fn(cid * HALF + r))


@functools.partial(
    pl.kernel, mesh=_mesh,
    out_type=jax.ShapeDtypeStruct((2, NP, 128), jnp.float32),
    scratch_types=[
        pltpu.VMEM((SCH, KCH), jnp.int32),
        pltpu.VMEM((KCH, 128), jnp.float32),
        pltpu.VMEM((OCH, 128), jnp.float32),
        pltpu.VMEM_SHARED((AROWS, 128), jnp.float32),
    ],
)
def _sc_counts(ei0_hbm, ei1_hbm, out_hbm, idx_v, ones_v, zb_v, acc):
    cid = lax.axis_index("c")
    sid = lax.axis_index("s")
    lo = cid * HALF
    _fill_vmem(ones_v, KCH, 128, 1.0)
    _zero_vmem(zb_v, OCH, 128)
    for which, ei_hbm in ((0, ei0_hbm), (1, ei1_hbm)):
        _zero_acc(acc, zb_v, sid)
        plsc.subcore_barrier()
        for st in range(NST):
            pltpu.sync_copy(ei_hbm.at[sid, st], idx_v)
            _localize(idx_v, lo)

            def body(c, _):
                pltpu.sync_copy(ones_v, acc.at[idx_v.at[c]], add=True)
                return 0
            lax.fori_loop(0, SCH, body, 0)
        plsc.subcore_barrier()
        _copy_out(acc, zb_v,
                  lambda r, w=which: out_hbm.at[w, pl.ds(r, OCH)], cid, sid)
        plsc.subcore_barrier()


@functools.lru_cache(maxsize=None)
def _sc_propagate(dp):
    @functools.partial(
        pl.kernel, mesh=_mesh,
        out_type=jax.ShapeDtypeStruct((NP, dp), jnp.float32),
        scratch_types=[
            pltpu.VMEM((SCH, KCH), jnp.int32),
            pltpu.VMEM((SCH, KCH), jnp.int32),
            pltpu.VMEM((KCH, dp), jnp.float32),
            pltpu.VMEM((KCH, dp), jnp.float32),
            pltpu.VMEM((OCH, dp), jnp.float32),
            pltpu.VMEM_SHARED((AROWS, dp), jnp.float32),
            pltpu.SemaphoreType.DMA,
            pltpu.SemaphoreType.DMA,
        ],
    )
    def k(xsrc_hbm, src_hbm, dst_hbm, out_hbm, sidx_v, didx_v, rows_a, rows_b,
          zb_v, acc, sem_a, sem_b):
        cid = lax.axis_index("c")
        sid = lax.axis_index("s")
        lo = cid * HALF
        _zero_vmem(zb_v, OCH, dp)
        _zero_acc(acc, zb_v, sid)
        plsc.subcore_barrier()

        def start(c, buf, sem):
            pltpu.make_async_copy(xsrc_hbm.at[sidx_v.at[c]], buf, sem).start()

        def finish(c, buf, sem):
            pltpu.make_async_copy(xsrc_hbm.at[sidx_v.at[c]], buf, sem).wait()
            pltpu.sync_copy(buf, acc.at[didx_v.at[c]], add=True)

        for st in range(NST):
            pltpu.sync_copy(src_hbm.at[sid, st], sidx_v)
            pltpu.sync_copy(dst_hbm.at[sid, st], didx_v)
            _localize(didx_v, lo)

            start(0, rows_a, sem_a)
            nb2 = SCH // 2

            def body(j, _):
                c0 = 2 * j
                start(c0 + 1, rows_b, sem_b)
                finish(c0, rows_a, sem_a)

                @pl.when(j < nb2 - 1)
                def _():
                    start(c0 + 2, rows_a, sem_a)

                finish(c0 + 1, rows_b, sem_b)
                return 0
            lax.fori_loop(0, nb2, body, 0)
            start(SCH - 1, rows_a, sem_a)
            finish(SCH - 1, rows_a, sem_a)

        plsc.subcore_barrier()
        _copy_out(acc, zb_v, lambda r: out_hbm.at[pl.ds(r, OCH)], cid, sid)
    return k


_MBLK = 1024


def _tc_matmul(x, w):
    n, k = x.shape
    o = w.shape[0]

    def body(x_ref, w_ref, o_ref):
        o_ref[...] = lax.dot_general(
            x_ref[...], w_ref[...], (((1,), (1,)), ((), ())),
            preferred_element_type=jnp.float32)

    return pl.pallas_call(
        body,
        grid=(n // _MBLK,),
        in_specs=[pl.BlockSpec((_MBLK, k), lambda i: (i, 0)),
                  pl.BlockSpec((o, k), lambda i: (0, 0))],
        out_specs=pl.BlockSpec((_MBLK, o), lambda i: (i, 0)),
        out_shape=jax.ShapeDtypeStruct((n, o), jnp.float32),
    )(x, w)


def _tc_combine(t, cnt, mode, bias=None, xl=None):
    dp = xl.shape[1] if t is None else t.shape[1]
    nin = []
    specs = []
    if t is not None:
        nin.append(t)
        specs.append(pl.BlockSpec((_MBLK, dp), lambda i: (i, 0)))
    nin.append(cnt)
    specs.append(pl.BlockSpec((_MBLK, L), lambda i: (i, 0)))
    if xl is not None:
        nin.append(xl)
        specs.append(pl.BlockSpec((_MBLK, dp), lambda i: (i, 0)))
    if bias is not None:
        nin.append(bias.reshape(1, dp))
        specs.append(pl.BlockSpec((1, dp), lambda i: (0, 0)))

    def body(*refs):
        refs = list(refs)
        out_ref = refs.pop()
        tv = refs.pop(0)[...] if t is not None else None
        cv = refs.pop(0)[...][:, 0:1]
        xv = refs.pop(0)[...] if xl is not None else None
        bv = refs.pop(0)[...] if bias is not None else None
        if mode == 'binv':
            out_ref[...] = jnp.where(cv > 0, tv / jnp.where(cv > 0, cv, 1.0), 0.0)
        elif mode == 'dinv':
            s = jnp.where(cv > 0, tv / jnp.where(cv > 0, cv, 1.0), 0.0)
            out_ref[...] = jnp.maximum(s + bv, 0.0)
        elif mode == 'dis':
            out_ref[...] = lax.rsqrt(cv + 1.0) * xv
        elif mode == 'gcn':
            d = lax.rsqrt(cv + 1.0)
            out_ref[...] = jnp.maximum(d * tv + d * d * xv + bv, 0.0)

    return pl.pallas_call(
        body,
        grid=(NP // _MBLK,),
        in_specs=specs,
        out_specs=pl.BlockSpec((_MBLK, dp), lambda i: (i, 0)),
        out_shape=jax.ShapeDtypeStruct((NP, dp), jnp.float32),
    )(*nin)


def _pad_w(w, o_t, k_t):
    return jnp.pad(w, ((0, o_t - w.shape[0]), (0, k_t - w.shape[1])))


def _head_body(xc_ref, w3_ref, b3_ref, g3_ref, bb3_ref, w4_ref, b4_ref,
               g4_ref, bb4_ref, w5_ref, b5_ref, wo_ref, bo_ref, out_ref):
    eps = 1e-5
    xc = xc_ref[...]
    h = jnp.maximum(xc @ w3_ref[...].T + b3_ref[...], 0.0)
    m = jnp.mean(h, axis=0, keepdims=True)
    v = jnp.mean((h - m) ** 2, axis=0, keepdims=True)
    h = (h - m) / jnp.sqrt(v + eps) * g3_ref[...] + bb3_ref[...]
    h = jnp.maximum(h @ w4_ref[...].T + b4_ref[...], 0.0)
    m = jnp.mean(h, axis=0, keepdims=True)
    v = jnp.mean((h - m) ** 2, axis=0, keepdims=True)
    h = (h - m) / jnp.sqrt(v + eps) * g4_ref[...] + bb4_ref[...]
    h = jnp.maximum(h @ w5_ref[...].T + b5_ref[...], 0.0)
    out_ref[...] = h @ wo_ref[...].T + bo_ref[...]


def _head(xc, p):
    return pl.pallas_call(
        _head_body,
        out_shape=jax.ShapeDtypeStruct((xc.shape[0], 2), jnp.float32),
    )(xc, p['fc_3_W'], p['fc_3_b'], p['n256_g'], p['n256_b'],
      p['fc_4_W'], p['fc_4_b'], p['n128_g'], p['n128_b'],
      p['fc_5_W'], p['fc_5_b'], p['out_W'], p['out_b'])


def _linear(x, W, b):
    return x @ W.T + b


def _ln(x, g, b, eps=1e-5):
    m = jnp.mean(x, axis=-1, keepdims=True)
    v = jnp.var(x, axis=-1, keepdims=True)
    return (x - m) / jnp.sqrt(v + eps) * g + b


def _bn(x, g, b, eps=1e-5):
    m = jnp.mean(x, axis=0)
    v = jnp.var(x, axis=0)
    return (x - m) / jnp.sqrt(v + eps) * g + b


def _mha(x, p, nh):
    B, Lq, D = x.shape
    hd = D // nh

    def split(t):
        return jnp.transpose(t.reshape(B, Lq, nh, hd), (0, 2, 1, 3))

    q = split(_linear(x, p['Wq'], p['bq']))
    k = split(_linear(x, p['Wk'], p['bk']))
    v = split(_linear(x, p['Wv'], p['bv']))
    s = jnp.einsum('bhqd,bhkd->bhqk', q, k) / math.sqrt(hd)
    a = jax.nn.softmax(s, axis=-1)
    o = jnp.einsum('bhqk,bhkd->bhqd', a, v)
    o = jnp.transpose(o, (0, 2, 1, 3)).reshape(B, Lq, D)
    return _linear(o, p['Wo'], p['bo'])


def _transformer(x, p, nh):
    tok = _linear(x, p['emb_W'], p['emb_b'])
    pe = jnp.sin(jnp.arange(x.shape[1], dtype=jnp.float32))[None, :, None]
    h = _ln(tok + (x + pe), p['embln_g'], p['embln_b'], eps=1e-12)
    a = _ln(h, p['ln1_g'], p['ln1_b'])
    h = h + _mha(a, p, nh)
    f = _ln(h, p['ln2_g'], p['ln2_b'])
    f = _linear(jax.nn.gelu(_linear(f, p['ff1_W'], p['ff1_b']), approximate=False),
                p['ff2_W'], p['ff2_b'])
    return h + f


def kernel(x, edge_index, batch, fingerprint, seqbefore, seqafter, variant, params):
    p = params
    src = edge_index[0].reshape(NS, NST, SCH, KCH)
    dst = edge_index[1].reshape(NS, NST, SCH, KCH)

    cparts = _sc_counts(src, dst)
    cnt_node = cparts[0, :, :L]
    cnt_he = cparts[1, :, :L]

    xpad = jnp.pad(x, ((0, NP - N), (0, 0)))

    w1 = _pad_w(p['hc1_W'], 128, XD)
    xl1 = _tc_matmul(xpad, w1)
    s1 = _sc_propagate(128)(xl1, src, dst)
    m1 = _tc_combine(s1, cnt_he, 'binv')
    t1 = _sc_propagate(128)(m1, dst, src)
    h1 = _tc_combine(t1, cnt_node, 'dinv',
                     bias=jnp.pad(p['hc1_b'], (0, 50)))

    w2 = _pad_w(p['hc2_W'], 384, 128)
    xl2 = _tc_matmul(h1, w2)
    b2 = jnp.pad(p['hc2_b'], (0, 72))
    h2panels = []
    for c0 in range(0, 384, 128):
        s2 = _sc_propagate(128)(xl2[:, c0:c0 + 128], src, dst)
        m2 = _tc_combine(s2, cnt_he, 'binv')
        t2 = _sc_propagate(128)(m2, dst, src)
        h2panels.append(_tc_combine(t2, cnt_node, 'dinv', bias=b2[c0:c0 + 128]))
    h2 = jnp.concatenate(h2panels, axis=1)

    w3 = _pad_w(p['gcn_W'], 896, 384)
    xl3 = _tc_matmul(h2, w3)
    b3 = jnp.pad(p['gcn_b'], (0, 116))
    h3panels = []
    for c0 in range(0, 896, 128):
        xl3p = xl3[:, c0:c0 + 128]
        zp = _tc_combine(None, cnt_he, 'dis', xl=xl3p)
        t3 = _sc_propagate(128)(zp, src, dst)
        h3panels.append(_tc_combine(t3, cnt_he, 'gcn', bias=b3[c0:c0 + 128],
                                    xl=xl3p))
    h = jnp.concatenate(h3panels, axis=1)[:N, :780]

    gmax = jax.ops.segment_max(h, batch, G)
    cnt = jnp.clip(jax.ops.segment_sum(jnp.ones((N,), jnp.float32), batch, G), 1.0, None)
    gmean = jax.ops.segment_sum(h, batch, G) / cnt[:, None]
    xg = jnp.concatenate([gmax, gmean], axis=1)
    xg = jax.nn.relu(_linear(xg, p['fc_g1_W'], p['fc_g1_b']))
    xg = _linear(xg, p['fc_g2_W'], p['fc_g2_b'])
    tf = _transformer(fingerprint[..., None], p['tf'], 8)
    flat_fc = jax.nn.relu(_linear(tf.reshape(tf.shape[0], -1), p['flat_fc_W'], p['flat_fc_b']))
    flat_fc = _bn(flat_fc, p['nxf_g'], p['nxf_b'])
    tsb = _transformer(seqbefore, p['tb'], 4)
    tsa = _transformer(seqafter, p['ta'], 4)
    fxsb = jax.nn.relu(_linear(tsb.reshape(tsb.shape[0], -1), p['flat_xs_W'], p['flat_xs_b']))
    fxsa = jax.nn.relu(_linear(tsa.reshape(tsa.shape[0], -1), p['flat_xs_W'], p['flat_xs_b']))
    fv = jax.nn.relu(_linear(variant, p['fc_xv_W'], p['fc_xv_b']))
    fv = _bn(fv, p['nxv_g'], p['nxv_b'])
    c1 = jnp.concatenate([flat_fc, fv], axis=1)
    c1 = _bn(jax.nn.relu(_linear(c1, p['fc2_1_W'], p['fc2_1_b'])), p['n1024_g'], p['n1024_b'])
    c2 = jnp.concatenate([fxsb, fxsa], axis=1)
    c2 = _bn(jax.nn.relu(_linear(c2, p['fc2_2_W'], p['fc2_2_b'])), p['n128_g'], p['n128_b'])
    xc = jnp.concatenate([xg, c1, c2], axis=1)
    return _head(xc, p)
